# baseline (speedup 1.0000x reference)
import functools

import jax
import jax.numpy as jnp
from jax import lax
from jax.experimental import pallas as pl
from jax.experimental.pallas import tpu as pltpu
from jax.experimental.pallas import tpu_sc as plsc

B = 16
N = 48
NP = 12
E = 768
ESP = 384
EPS = 384
IN = 16
MSG = 16
BOND = 12
HID = 32
OUT = 16
F = 2 * IN + BOND

NS = 16
G = 2
EC = E // NS
EC_SP = ESP // NS
EC_PS = EPS // NS
EALL = EC + EC_SP + EC_PS
SPT = N // NS

_S_BASE = 0
_SP_BASE = B * N * IN

_ACC_M0 = 0
_ACC_PS0 = N * MSG
_ACC_SP0 = 2 * N * MSG
_ACC_ROWS = 2 * N * MSG + NP * MSG
_ACC_W = _ACC_ROWS * B
_NRED = 4
_HW = _ACC_W // _NRED
_CHUNK = _HW // NS

_W_MSG0 = 0
_W_PS0 = MSG * F
_W_SP0 = 2 * MSG * F
_WEDGE_ROWS = 3 * MSG * F
_WN1_0 = 0
_WN2_0 = HID * (IN + 2 * MSG)
_WN_ROWS = _WN2_0 + OUT * HID
_WP1_0 = 0
_WP2_0 = HID * (IN + MSG)
_WP_ROWS = _WP2_0 + OUT * HID

_B_MSG, _AW_MSG, _AB_MSG = 0, 16, 32
_B_PS, _AW_PS, _AB_PS = 33, 49, 65
_B_SP, _AW_SP, _AB_SP = 66, 82, 98
_BN1_0, _BN2_0, _BP1_0, _BP2_0 = 99, 131, 147, 179
_BIAS_ROWS = 195

_XF_X0, _XF_M0, _XF_P0, _XF_H0 = 0, 16, 32, 48
_XF_ROWS = 80

_f32 = jnp.float32
_i32 = jnp.int32


def _leaky(x):
    return jnp.maximum(x, 0.01 * x)


def _sigmoid(x):
    return 1.0 / (1.0 + jnp.exp(-x))


def _bcast_i(x):
    return jnp.full((B,), x, dtype=_i32)


def _edge_loop(biota, pack_v, bonds_v, allsites_v, wbuf, bias_v, acc_ref):
    def body(t, carry):
        pk = pack_v[pl.ds(t * B, B)]
        soff = [pk[g] for g in range(G)]
        roff = [pk[G + g] for g in range(G)]
        accw = [pk[2 * G + g] for g in range(G)]
        bs = biota * pk[6]
        br = biota * pk[7]
        w0 = pk[8]
        b0 = pk[9]
        aw0 = pk[10]
        ab0 = pk[11]

        accs = tuple(bias_v[pl.ds((b0 + o) * B, B)]
                     for _ in range(G) for o in range(MSG))

        UNR = 2

        def gather_sub(col0, offs, bvec):
            def fb(ff, accs):
                accs = list(accs)
                for u in range(UNR):
                    f = ff * UNR + u
                    vs = [plsc.load_gather(
                        allsites_v, [bvec + _bcast_i(offs[g] + f)])
                        for g in range(G)]
                    wbase = (w0 + col0 + f) * B
                    for o in range(MSG):
                        w = wbuf[pl.ds(wbase + o * (F * B), B)]
                        for g in range(G):
                            accs[g * MSG + o] = accs[g * MSG + o] + w * vs[g]
                return tuple(accs)
            return fb

        accs = lax.fori_loop(0, IN // UNR, gather_sub(0, soff, bs), accs)
        accs = lax.fori_loop(0, IN // UNR, gather_sub(IN, roff, br), accs)
        eb = [(G * t + g) * (BOND * B) for g in range(G)]

        def bond_fb(ff, accs):
            accs = list(accs)
            for u in range(UNR):
                f = ff * UNR + u
                vs = [bonds_v[pl.ds(eb[g] + f * B, B)] for g in range(G)]
                wbase = (w0 + 2 * IN + f) * B
                for o in range(MSG):
                    w = wbuf[pl.ds(wbase + o * (F * B), B)]
                    for g in range(G):
                        accs[g * MSG + o] = accs[g * MSG + o] + w * vs[g]
            return tuple(accs)

        accs = lax.fori_loop(0, BOND // UNR, bond_fb, accs)

        for g in range(G):
            acc = [_leaky(accs[g * MSG + o]) for o in range(MSG)]
            s = bias_v[pl.ds(ab0 * B, B)]
            for o in range(MSG):
                s = s + bias_v[pl.ds((aw0 + o) * B, B)] * acc[o]
            att = _sigmoid(s)
            for o in range(MSG):
                plsc.addupdate_scatter(
                    acc_ref, [_bcast_i(accw[g] + o * B) + biota],
                    att * acc[o])
        return carry

    plsc.parallel_loop(0, EALL // G, step=1, carry=jnp.int32(0))(body)


def _mlp_site_loop(n_sites, biota, base, x_base, x_stride, m_off, p_off,
                   two_blocks, allsites_v, sh_red, xfeat_v, wbuf, w1_0, w2_0,
                   bias_v, b1_0, b2_0, nfeat, osites_v):
    bx = biota * x_stride

    def site_body(j, carry):
        n = base + j
        for k in range(IN):
            xfeat_v[pl.ds((_XF_X0 + k) * B, B)] = plsc.load_gather(
                allsites_v, [bx + _bcast_i(x_base + n * IN + k)])
        pltpu.sync_copy(sh_red.at[pl.ds((m_off + n * MSG) * B, MSG * B)],
                        xfeat_v.at[pl.ds(_XF_M0 * B, MSG * B)])
        if two_blocks:
            pltpu.sync_copy(sh_red.at[pl.ds((p_off + n * MSG) * B, MSG * B)],
                            xfeat_v.at[pl.ds(_XF_P0 * B, MSG * B)])

        for half in range(HID // MSG):
            accs = tuple(bias_v[pl.ds((b1_0 + half * MSG + o) * B, B)]
                         for o in range(MSG))

            def l1_body(k, accs):
                xk = xfeat_v[pl.ds(k * B, B)]
                accs = list(accs)
                for o in range(MSG):
                    w = wbuf[pl.ds(
                        (w1_0 + (half * MSG + o) * nfeat + k) * B, B)]
                    accs[o] = accs[o] + w * xk
                return tuple(accs)

            accs = lax.fori_loop(0, nfeat, l1_body, accs)
            for jj in range(MSG):
                xfeat_v[pl.ds((_XF_H0 + half * MSG + jj) * B, B)] = _leaky(
                    accs[jj])

        outs = tuple(bias_v[pl.ds((b2_0 + o) * B, B)] for o in range(OUT))

        def l2_body(jh, outs):
            hv = xfeat_v[pl.ds((_XF_H0 + jh) * B, B)]
            outs = list(outs)
            for o in range(OUT):
                w = wbuf[pl.ds((w2_0 + o * HID + jh) * B, B)]
                outs[o] = outs[o] + w * hv
            return tuple(outs)

        outs = lax.fori_loop(0, HID, l2_body, outs)
        for o in range(OUT):
            osites_v[pl.ds((j * OUT + o) * B, B)] = (
                xfeat_v[pl.ds((_XF_X0 + o) * B, B)] + _leaky(outs[o]))
        return carry

    lax.fori_loop(0, n_sites, site_body, 0)


def _sc_body(sites_h, sitesp_h, bonds_h, bonds_sp_h, bonds_ps_h, pack_h,
             wedge_h, wn_h, wp_h, bias_h, zeros_h,
             sites_out_h, sites_p_out_h,
             allsites_v, bonds_v, pack_v, wbuf, bias_v,
             acc_v, redin_v, redout_v,
             xfeat_v, osites_v, ositesp_v,
             sh_slots, sh_red):
    sid = lax.axis_index("s")
    biota = lax.iota(_i32, B)

    pltpu.sync_copy(sites_h, allsites_v.at[pl.ds(_S_BASE, B * N * IN)])
    pltpu.sync_copy(sitesp_h, allsites_v.at[pl.ds(_SP_BASE, B * NP * IN)])
    pltpu.sync_copy(pack_h.at[pl.ds(sid * (EALL // G) * B, (EALL // G) * B)],
                    pack_v)
    pltpu.sync_copy(wedge_h, wbuf.at[pl.ds(0, _WEDGE_ROWS * B)])
    pltpu.sync_copy(bias_h, bias_v)
    pltpu.sync_copy(zeros_h, acc_v)
    pltpu.sync_copy(bonds_h.at[pl.ds(sid * EC * BOND * B, EC * BOND * B)],
                    bonds_v.at[pl.ds(0, EC * BOND * B)])
    pltpu.sync_copy(
        bonds_ps_h.at[pl.ds(sid * EC_PS * BOND * B, EC_PS * BOND * B)],
        bonds_v.at[pl.ds(EC * BOND * B, EC_PS * BOND * B)])
    pltpu.sync_copy(
        bonds_sp_h.at[pl.ds(sid * EC_SP * BOND * B, EC_SP * BOND * B)],
        bonds_v.at[pl.ds((EC + EC_PS) * BOND * B, EC_SP * BOND * B)])

    _edge_loop(biota, pack_v, bonds_v, allsites_v, wbuf, bias_v, acc_v)

    for h in range(_NRED):
        pltpu.sync_copy(acc_v.at[pl.ds(h * _HW, _HW)],
                        sh_slots.at[pl.ds(sid * _HW, _HW)])
        plsc.subcore_barrier()
        for k in range(NS):
            pltpu.sync_copy(
                sh_slots.at[pl.ds(k * _HW + sid * _CHUNK, _CHUNK)],
                redin_v.at[pl.ds(k * _CHUNK, _CHUNK)])

        def red_body(r, carry):
            s = redin_v[pl.ds(r * B, B)]
            for k in range(1, NS):
                s = s + redin_v[pl.ds(k * _CHUNK + r * B, B)]
            redout_v[pl.ds(r * B, B)] = s
            return carry

        plsc.parallel_loop(0, _CHUNK // B, step=1,
                           carry=jnp.int32(0))(red_body)
        pltpu.sync_copy(redout_v,
                        sh_red.at[pl.ds(h * _HW + sid * _CHUNK, _CHUNK)])
        plsc.subcore_barrier()

    pltpu.sync_copy(wn_h, wbuf.at[pl.ds(0, _WN_ROWS * B)])
    _mlp_site_loop(SPT, biota, sid * SPT, _S_BASE, N * IN,
                   _ACC_M0, _ACC_PS0, True, allsites_v, sh_red, xfeat_v,
                   wbuf, _WN1_0, _WN2_0, bias_v, _BN1_0, _BN2_0,
                   IN + 2 * MSG, osites_v)
    pltpu.sync_copy(osites_v,
                    sites_out_h.at[pl.ds(sid * SPT * OUT * B, SPT * OUT * B)])

    @pl.when(sid < NP)
    def _():
        pltpu.sync_copy(wp_h, wbuf.at[pl.ds(0, _WP_ROWS * B)])
        _mlp_site_loop(1, biota, sid, _SP_BASE, NP * IN,
                       _ACC_SP0, _ACC_SP0, False, allsites_v, sh_red,
                       xfeat_v, wbuf, _WP1_0, _WP2_0, bias_v,
                       _BP1_0, _BP2_0, IN + MSG, ositesp_v)
        pltpu.sync_copy(ositesp_v,
                        sites_p_out_h.at[pl.ds(sid * OUT * B, OUT * B)])


_mesh = plsc.VectorSubcoreMesh(core_axis_name="c", subcore_axis_name="s",
                               num_cores=1)

_sc_call = functools.partial(
    pl.kernel,
    out_type=[
        jax.ShapeDtypeStruct((N * OUT * B,), _f32),
        jax.ShapeDtypeStruct((NP * OUT * B,), _f32),
    ],
    mesh=_mesh,
    compiler_params=pltpu.CompilerParams(needs_layout_passes=False),
    scratch_types=[
        pltpu.VMEM((B * (N + NP) * IN,), _f32),
        pltpu.VMEM((EALL * BOND * B,), _f32),
        pltpu.VMEM(((EALL // G) * B,), _i32),
        pltpu.VMEM((_WEDGE_ROWS * B,), _f32),
        pltpu.VMEM((_BIAS_ROWS * B,), _f32),
        pltpu.VMEM((_ACC_W,), _f32),
        pltpu.VMEM((NS * _CHUNK,), _f32),
        pltpu.VMEM((_CHUNK,), _f32),
        pltpu.VMEM((_XF_ROWS * B,), _f32),
        pltpu.VMEM((SPT * OUT * B,), _f32),
        pltpu.VMEM((OUT * B,), _f32),
        pltpu.VMEM_SHARED((NS * _HW,), _f32),
        pltpu.VMEM_SHARED((_ACC_W,), _f32),
    ],
)(_sc_body)


def _seg_pack(i1, i2, s_base, r_base, acc_base, stride_s, stride_r,
              w0, b0, aw0, ab0):
    n = i1.shape[0] // G
    c = [s_base + i1.reshape(-1, G) * IN,
         r_base + i2.reshape(-1, G) * IN,
         (acc_base + i2.reshape(-1, G) * MSG) * B]
    const = jnp.broadcast_to(
        jnp.array([stride_s, stride_r, w0, b0, aw0, ab0]
                  + [0] * (B - 3 * G - 6), _i32), (n, B - 3 * G))
    p = jnp.concatenate(c + [const], axis=1)
    return p.reshape(NS, n // NS, B)


@jax.jit
def _run(sites, bonds, sites_p, bonds_sp, bonds_ps,
         idx1, idx2, idx1_sp, idx2_sp, idx1_ps, idx2_ps,
         W_msg, b_msg, aW_msg, ab_msg,
         W_sp, b_sp, aW_sp, ab_sp,
         W_ps, b_ps, aW_ps, ab_ps,
         Wn1, bn1, Wn2, bn2, Wp1, bp1, Wp2, bp2):
    wedge = jnp.concatenate([W_msg.reshape(-1), W_ps.reshape(-1),
                             W_sp.reshape(-1)])
    wn = jnp.concatenate([Wn1.reshape(-1), Wn2.reshape(-1)])
    wp = jnp.concatenate([Wp1.reshape(-1), Wp2.reshape(-1)])
    bias = jnp.concatenate([
        b_msg, aW_msg.reshape(-1), ab_msg,
        b_ps, aW_ps.reshape(-1), ab_ps,
        b_sp, aW_sp.reshape(-1), ab_sp,
        bn1, bn2, bp1, bp2])
    bcast = lambda v: jnp.broadcast_to(
        v.reshape(-1, 1), (v.shape[0], B)).reshape(-1)
    tflat = lambda a: a.transpose(1, 2, 0).reshape(-1)
    pack = jnp.concatenate([
        _seg_pack(idx1, idx2, _S_BASE, _S_BASE, _ACC_M0, N * IN, N * IN,
                  _W_MSG0, _B_MSG, _AW_MSG, _AB_MSG),
        _seg_pack(idx1_ps, idx2_ps, _SP_BASE, _S_BASE, _ACC_PS0, NP * IN,
                  N * IN, _W_PS0, _B_PS, _AW_PS, _AB_PS),
        _seg_pack(idx1_sp, idx2_sp, _S_BASE, _SP_BASE, _ACC_SP0, N * IN,
                  NP * IN, _W_SP0, _B_SP, _AW_SP, _AB_SP),
    ], axis=1).reshape(-1)
    return _sc_call(
        sites.reshape(-1), sites_p.reshape(-1),
        tflat(bonds), tflat(bonds_sp), tflat(bonds_ps), pack,
        bcast(wedge), bcast(wn), bcast(wp), bcast(bias),
        jnp.zeros((_ACC_W,), _f32))


def kernel(sites, bonds, sites_p, bonds_sp, bonds_ps,
           idx1, idx2, idx1_sp, idx2_sp, idx1_ps, idx2_ps,
           W_msg, b_msg, aW_msg, ab_msg,
           W_sp, b_sp, aW_sp, ab_sp,
           W_ps, b_ps, aW_ps, ab_ps,
           Wn1, bn1, Wn2, bn2, Wp1, bp1, Wp2, bp2):
    i32 = lambda x: x.astype(_i32)
    sites_new_t, sites_p_new_t = _run(
        sites, bonds, sites_p, bonds_sp, bonds_ps,
        i32(idx1), i32(idx2), i32(idx1_sp), i32(idx2_sp),
        i32(idx1_ps), i32(idx2_ps),
        W_msg, b_msg, aW_msg, ab_msg,
        W_sp, b_sp, aW_sp, ab_sp,
        W_ps, b_ps, aW_ps, ab_ps,
        Wn1, bn1, Wn2, bn2, Wp1, bp1, Wp2, bp2)
    sites_new = sites_new_t.reshape(N, OUT, B).transpose(2, 0, 1)
    sites_p_new = sites_p_new_t.reshape(NP, OUT, B).transpose(2, 0, 1)
    return (sites_new, bonds, sites_p_new, bonds_sp, bonds_ps)


# speedup vs baseline: 1.0296x; 1.0296x over previous
import functools

import jax
import jax.numpy as jnp
from jax import lax
from jax.experimental import pallas as pl
from jax.experimental.pallas import tpu as pltpu
from jax.experimental.pallas import tpu_sc as plsc

B = 16
N = 48
NP = 12
E = 768
ESP = 384
EPS = 384
IN = 16
MSG = 16
BOND = 12
HID = 32
OUT = 16
F = 2 * IN + BOND

NS = 16
G = 2
EC = E // NS
EC_SP = ESP // NS
EC_PS = EPS // NS
EALL = EC + EC_SP + EC_PS
SPT = N // NS

_S_BASE = 0
_SP_BASE = B * N * IN

_ACC_M0 = 0
_ACC_PS0 = N * MSG
_ACC_SP0 = 2 * N * MSG
_ACC_ROWS = 2 * N * MSG + NP * MSG
_ACC_W = _ACC_ROWS * B
_NRED = 4
_HW = _ACC_W // _NRED
_CHUNK = _HW // NS

_W_MSG0 = 0
_W_PS0 = MSG * F
_W_SP0 = 2 * MSG * F
_WEDGE_ROWS = 3 * MSG * F
_WN1_0 = 0
_WN2_0 = HID * (IN + 2 * MSG)
_WN_ROWS = _WN2_0 + OUT * HID
_WP1_0 = 0
_WP2_0 = HID * (IN + MSG)
_WP_ROWS = _WP2_0 + OUT * HID

_B_MSG, _AW_MSG, _AB_MSG = 0, 16, 32
_B_PS, _AW_PS, _AB_PS = 33, 49, 65
_B_SP, _AW_SP, _AB_SP = 66, 82, 98
_BN1_0, _BN2_0, _BP1_0, _BP2_0 = 99, 131, 147, 179
_BIAS_ROWS = 195

_XF_X0, _XF_M0, _XF_P0, _XF_H0 = 0, 16, 32, 48
_XF_ROWS = 80

_f32 = jnp.float32
_i32 = jnp.int32


def _leaky(x):
    return jnp.maximum(x, 0.01 * x)


def _sigmoid(x):
    return 1.0 / (1.0 + jnp.exp(-x))


def _bcast_i(x):
    return jnp.full((B,), x, dtype=_i32)


def _edge_loop(biota, pack_v, bonds_v, allsites_v, wbuf, bias_v, acc_ref):
    def body(t, carry):
        pk = pack_v[pl.ds(t * B, B)]
        soff = [pk[g] for g in range(G)]
        roff = [pk[G + g] for g in range(G)]
        accw = [pk[2 * G + g] for g in range(G)]
        bs = biota * pk[6]
        br = biota * pk[7]
        w0 = pk[8]
        b0 = pk[9]
        aw0 = pk[10]
        ab0 = pk[11]

        accs = tuple(bias_v[pl.ds((b0 + o) * B, B)]
                     for _ in range(G) for o in range(MSG))

        UNR = 1

        def gather_sub(col0, offs, bvec):
            def fb(ff, accs):
                accs = list(accs)
                for u in range(UNR):
                    f = ff * UNR + u
                    vs = [plsc.load_gather(
                        allsites_v, [bvec + _bcast_i(offs[g] + f)])
                        for g in range(G)]
                    wbase = (w0 + col0 + f) * B
                    for o in range(MSG):
                        w = wbuf[pl.ds(wbase + o * (F * B), B)]
                        for g in range(G):
                            accs[g * MSG + o] = accs[g * MSG + o] + w * vs[g]
                return tuple(accs)
            return fb

        accs = lax.fori_loop(0, IN // UNR, gather_sub(0, soff, bs), accs)
        accs = lax.fori_loop(0, IN // UNR, gather_sub(IN, roff, br), accs)
        eb = [(G * t + g) * (BOND * B) for g in range(G)]

        def bond_fb(ff, accs):
            accs = list(accs)
            for u in range(UNR):
                f = ff * UNR + u
                vs = [bonds_v[pl.ds(eb[g] + f * B, B)] for g in range(G)]
                wbase = (w0 + 2 * IN + f) * B
                for o in range(MSG):
                    w = wbuf[pl.ds(wbase + o * (F * B), B)]
                    for g in range(G):
                        accs[g * MSG + o] = accs[g * MSG + o] + w * vs[g]
            return tuple(accs)

        accs = lax.fori_loop(0, BOND // UNR, bond_fb, accs)

        for g in range(G):
            acc = [_leaky(accs[g * MSG + o]) for o in range(MSG)]
            s = bias_v[pl.ds(ab0 * B, B)]
            for o in range(MSG):
                s = s + bias_v[pl.ds((aw0 + o) * B, B)] * acc[o]
            att = _sigmoid(s)
            for o in range(MSG):
                plsc.addupdate_scatter(
                    acc_ref, [_bcast_i(accw[g] + o * B) + biota],
                    att * acc[o])
        return carry

    plsc.parallel_loop(0, EALL // G, step=1, carry=jnp.int32(0))(body)


def _mlp_site_loop(n_sites, biota, base, x_base, x_stride, m_off, p_off,
                   two_blocks, allsites_v, sh_red, xfeat_v, wbuf, w1_0, w2_0,
                   bias_v, b1_0, b2_0, nfeat, osites_v):
    bx = biota * x_stride

    def site_body(j, carry):
        n = base + j
        for k in range(IN):
            xfeat_v[pl.ds((_XF_X0 + k) * B, B)] = plsc.load_gather(
                allsites_v, [bx + _bcast_i(x_base + n * IN + k)])
        pltpu.sync_copy(sh_red.at[pl.ds((m_off + n * MSG) * B, MSG * B)],
                        xfeat_v.at[pl.ds(_XF_M0 * B, MSG * B)])
        if two_blocks:
            pltpu.sync_copy(sh_red.at[pl.ds((p_off + n * MSG) * B, MSG * B)],
                            xfeat_v.at[pl.ds(_XF_P0 * B, MSG * B)])

        for half in range(HID // MSG):
            accs = tuple(bias_v[pl.ds((b1_0 + half * MSG + o) * B, B)]
                         for o in range(MSG))

            def l1_body(k, accs):
                xk = xfeat_v[pl.ds(k * B, B)]
                accs = list(accs)
                for o in range(MSG):
                    w = wbuf[pl.ds(
                        (w1_0 + (half * MSG + o) * nfeat + k) * B, B)]
                    accs[o] = accs[o] + w * xk
                return tuple(accs)

            accs = lax.fori_loop(0, nfeat, l1_body, accs)
            for jj in range(MSG):
                xfeat_v[pl.ds((_XF_H0 + half * MSG + jj) * B, B)] = _leaky(
                    accs[jj])

        outs = tuple(bias_v[pl.ds((b2_0 + o) * B, B)] for o in range(OUT))

        def l2_body(jh, outs):
            hv = xfeat_v[pl.ds((_XF_H0 + jh) * B, B)]
            outs = list(outs)
            for o in range(OUT):
                w = wbuf[pl.ds((w2_0 + o * HID + jh) * B, B)]
                outs[o] = outs[o] + w * hv
            return tuple(outs)

        outs = lax.fori_loop(0, HID, l2_body, outs)
        for o in range(OUT):
            osites_v[pl.ds((j * OUT + o) * B, B)] = (
                xfeat_v[pl.ds((_XF_X0 + o) * B, B)] + _leaky(outs[o]))
        return carry

    lax.fori_loop(0, n_sites, site_body, 0)


def _sc_body(sites_h, sitesp_h, bonds_h, bonds_sp_h, bonds_ps_h, pack_h,
             wedge_h, wn_h, wp_h, bias_h, zeros_h,
             sites_out_h, sites_p_out_h,
             allsites_v, bonds_v, pack_v, wbuf, bias_v,
             acc_v, redin_v, redout_v,
             xfeat_v, osites_v, ositesp_v,
             sh_slots, sh_red):
    sid = lax.axis_index("s")
    biota = lax.iota(_i32, B)

    pltpu.sync_copy(sites_h, allsites_v.at[pl.ds(_S_BASE, B * N * IN)])
    pltpu.sync_copy(sitesp_h, allsites_v.at[pl.ds(_SP_BASE, B * NP * IN)])
    pltpu.sync_copy(pack_h.at[pl.ds(sid * (EALL // G) * B, (EALL // G) * B)],
                    pack_v)
    pltpu.sync_copy(wedge_h, wbuf.at[pl.ds(0, _WEDGE_ROWS * B)])
    pltpu.sync_copy(bias_h, bias_v)
    pltpu.sync_copy(zeros_h, acc_v)
    pltpu.sync_copy(bonds_h.at[pl.ds(sid * EC * BOND * B, EC * BOND * B)],
                    bonds_v.at[pl.ds(0, EC * BOND * B)])
    pltpu.sync_copy(
        bonds_ps_h.at[pl.ds(sid * EC_PS * BOND * B, EC_PS * BOND * B)],
        bonds_v.at[pl.ds(EC * BOND * B, EC_PS * BOND * B)])
    pltpu.sync_copy(
        bonds_sp_h.at[pl.ds(sid * EC_SP * BOND * B, EC_SP * BOND * B)],
        bonds_v.at[pl.ds((EC + EC_PS) * BOND * B, EC_SP * BOND * B)])

    _edge_loop(biota, pack_v, bonds_v, allsites_v, wbuf, bias_v, acc_v)

    for h in range(_NRED):
        pltpu.sync_copy(acc_v.at[pl.ds(h * _HW, _HW)],
                        sh_slots.at[pl.ds(sid * _HW, _HW)])
        plsc.subcore_barrier()
        for k in range(NS):
            pltpu.sync_copy(
                sh_slots.at[pl.ds(k * _HW + sid * _CHUNK, _CHUNK)],
                redin_v.at[pl.ds(k * _CHUNK, _CHUNK)])

        def red_body(r, carry):
            s = redin_v[pl.ds(r * B, B)]
            for k in range(1, NS):
                s = s + redin_v[pl.ds(k * _CHUNK + r * B, B)]
            redout_v[pl.ds(r * B, B)] = s
            return carry

        plsc.parallel_loop(0, _CHUNK // B, step=1,
                           carry=jnp.int32(0))(red_body)
        pltpu.sync_copy(redout_v,
                        sh_red.at[pl.ds(h * _HW + sid * _CHUNK, _CHUNK)])
        plsc.subcore_barrier()

    pltpu.sync_copy(wn_h, wbuf.at[pl.ds(0, _WN_ROWS * B)])
    _mlp_site_loop(SPT, biota, sid * SPT, _S_BASE, N * IN,
                   _ACC_M0, _ACC_PS0, True, allsites_v, sh_red, xfeat_v,
                   wbuf, _WN1_0, _WN2_0, bias_v, _BN1_0, _BN2_0,
                   IN + 2 * MSG, osites_v)
    pltpu.sync_copy(osites_v,
                    sites_out_h.at[pl.ds(sid * SPT * OUT * B, SPT * OUT * B)])

    @pl.when(sid < NP)
    def _():
        pltpu.sync_copy(wp_h, wbuf.at[pl.ds(0, _WP_ROWS * B)])
        _mlp_site_loop(1, biota, sid, _SP_BASE, NP * IN,
                       _ACC_SP0, _ACC_SP0, False, allsites_v, sh_red,
                       xfeat_v, wbuf, _WP1_0, _WP2_0, bias_v,
                       _BP1_0, _BP2_0, IN + MSG, ositesp_v)
        pltpu.sync_copy(ositesp_v,
                        sites_p_out_h.at[pl.ds(sid * OUT * B, OUT * B)])


_mesh = plsc.VectorSubcoreMesh(core_axis_name="c", subcore_axis_name="s",
                               num_cores=1)

_sc_call = functools.partial(
    pl.kernel,
    out_type=[
        jax.ShapeDtypeStruct((N * OUT * B,), _f32),
        jax.ShapeDtypeStruct((NP * OUT * B,), _f32),
    ],
    mesh=_mesh,
    compiler_params=pltpu.CompilerParams(needs_layout_passes=False),
    scratch_types=[
        pltpu.VMEM((B * (N + NP) * IN,), _f32),
        pltpu.VMEM((EALL * BOND * B,), _f32),
        pltpu.VMEM(((EALL // G) * B,), _i32),
        pltpu.VMEM((_WEDGE_ROWS * B,), _f32),
        pltpu.VMEM((_BIAS_ROWS * B,), _f32),
        pltpu.VMEM((_ACC_W,), _f32),
        pltpu.VMEM((NS * _CHUNK,), _f32),
        pltpu.VMEM((_CHUNK,), _f32),
        pltpu.VMEM((_XF_ROWS * B,), _f32),
        pltpu.VMEM((SPT * OUT * B,), _f32),
        pltpu.VMEM((OUT * B,), _f32),
        pltpu.VMEM_SHARED((NS * _HW,), _f32),
        pltpu.VMEM_SHARED((_ACC_W,), _f32),
    ],
)(_sc_body)


def _seg_pack(i1, i2, s_base, r_base, acc_base, stride_s, stride_r,
              w0, b0, aw0, ab0):
    n = i1.shape[0] // G
    c = [s_base + i1.reshape(-1, G) * IN,
         r_base + i2.reshape(-1, G) * IN,
         (acc_base + i2.reshape(-1, G) * MSG) * B]
    const = jnp.broadcast_to(
        jnp.array([stride_s, stride_r, w0, b0, aw0, ab0]
                  + [0] * (B - 3 * G - 6), _i32), (n, B - 3 * G))
    p = jnp.concatenate(c + [const], axis=1)
    return p.reshape(NS, n // NS, B)


@jax.jit
def _run(sites, bonds, sites_p, bonds_sp, bonds_ps,
         idx1, idx2, idx1_sp, idx2_sp, idx1_ps, idx2_ps,
         W_msg, b_msg, aW_msg, ab_msg,
         W_sp, b_sp, aW_sp, ab_sp,
         W_ps, b_ps, aW_ps, ab_ps,
         Wn1, bn1, Wn2, bn2, Wp1, bp1, Wp2, bp2):
    wedge = jnp.concatenate([W_msg.reshape(-1), W_ps.reshape(-1),
                             W_sp.reshape(-1)])
    wn = jnp.concatenate([Wn1.reshape(-1), Wn2.reshape(-1)])
    wp = jnp.concatenate([Wp1.reshape(-1), Wp2.reshape(-1)])
    bias = jnp.concatenate([
        b_msg, aW_msg.reshape(-1), ab_msg,
        b_ps, aW_ps.reshape(-1), ab_ps,
        b_sp, aW_sp.reshape(-1), ab_sp,
        bn1, bn2, bp1, bp2])
    bcast = lambda v: jnp.broadcast_to(
        v.reshape(-1, 1), (v.shape[0], B)).reshape(-1)
    tflat = lambda a: a.transpose(1, 2, 0).reshape(-1)
    pack = jnp.concatenate([
        _seg_pack(idx1, idx2, _S_BASE, _S_BASE, _ACC_M0, N * IN, N * IN,
                  _W_MSG0, _B_MSG, _AW_MSG, _AB_MSG),
        _seg_pack(idx1_ps, idx2_ps, _SP_BASE, _S_BASE, _ACC_PS0, NP * IN,
                  N * IN, _W_PS0, _B_PS, _AW_PS, _AB_PS),
        _seg_pack(idx1_sp, idx2_sp, _S_BASE, _SP_BASE, _ACC_SP0, N * IN,
                  NP * IN, _W_SP0, _B_SP, _AW_SP, _AB_SP),
    ], axis=1).reshape(-1)
    return _sc_call(
        sites.reshape(-1), sites_p.reshape(-1),
        tflat(bonds), tflat(bonds_sp), tflat(bonds_ps), pack,
        bcast(wedge), bcast(wn), bcast(wp), bcast(bias),
        jnp.zeros((_ACC_W,), _f32))


def kernel(sites, bonds, sites_p, bonds_sp, bonds_ps,
           idx1, idx2, idx1_sp, idx2_sp, idx1_ps, idx2_ps,
           W_msg, b_msg, aW_msg, ab_msg,
           W_sp, b_sp, aW_sp, ab_sp,
           W_ps, b_ps, aW_ps, ab_ps,
           Wn1, bn1, Wn2, bn2, Wp1, bp1, Wp2, bp2):
    i32 = lambda x: x.astype(_i32)
    sites_new_t, sites_p_new_t = _run(
        sites, bonds, sites_p, bonds_sp, bonds_ps,
        i32(idx1), i32(idx2), i32(idx1_sp), i32(idx2_sp),
        i32(idx1_ps), i32(idx2_ps),
        W_msg, b_msg, aW_msg, ab_msg,
        W_sp, b_sp, aW_sp, ab_sp,
        W_ps, b_ps, aW_ps, ab_ps,
        Wn1, bn1, Wn2, bn2, Wp1, bp1, Wp2, bp2)
    sites_new = sites_new_t.reshape(N, OUT, B).transpose(2, 0, 1)
    sites_p_new = sites_p_new_t.reshape(NP, OUT, B).transpose(2, 0, 1)
    return (sites_new, bonds, sites_p_new, bonds_sp, bonds_ps)


# speedup vs baseline: 1.0644x; 1.0337x over previous
import functools

import jax
import jax.numpy as jnp
from jax import lax
from jax.experimental import pallas as pl
from jax.experimental.pallas import tpu as pltpu
from jax.experimental.pallas import tpu_sc as plsc

B = 16
N = 48
NP = 12
E = 768
ESP = 384
EPS = 384
IN = 16
MSG = 16
BOND = 12
HID = 32
OUT = 16
F = 2 * IN + BOND

NS = 16
G = 2
EC = E // NS
EC_SP = ESP // NS
EC_PS = EPS // NS
EALL = EC + EC_SP + EC_PS
SPT = N // NS

_S_BASE = 0
_SP_BASE = B * N * IN

_ACC_M0 = 0
_ACC_PS0 = N * MSG
_ACC_SP0 = 2 * N * MSG
_ACC_ROWS = 2 * N * MSG + NP * MSG
_ACC_W = _ACC_ROWS * B
_NRED = 4
_HW = _ACC_W // _NRED
_CHUNK = _HW // NS

_W_MSG0 = 0
_W_PS0 = MSG * F
_W_SP0 = 2 * MSG * F
_WEDGE_ROWS = 3 * MSG * F
_WN1_0 = 0
_WN2_0 = HID * (IN + 2 * MSG)
_WN_ROWS = _WN2_0 + OUT * HID
_WP1_0 = 0
_WP2_0 = HID * (IN + MSG)
_WP_ROWS = _WP2_0 + OUT * HID

_B_MSG, _AW_MSG, _AB_MSG = 0, 16, 32
_B_PS, _AW_PS, _AB_PS = 33, 49, 65
_B_SP, _AW_SP, _AB_SP = 66, 82, 98
_BN1_0, _BN2_0, _BP1_0, _BP2_0 = 99, 131, 147, 179
_BIAS_ROWS = 195

_WN_IN0 = _WEDGE_ROWS
_WP_IN0 = _WN_IN0 + _WN_ROWS
_BIAS0 = _WP_IN0 + _WP_ROWS

_XF_X0, _XF_M0, _XF_P0, _XF_H0 = 0, 16, 32, 48
_XF_ROWS = 80

_f32 = jnp.float32
_i32 = jnp.int32


def _leaky(x):
    return jnp.maximum(x, 0.01 * x)


def _sigmoid(x):
    return 1.0 / (1.0 + jnp.exp(-x))


def _bcast_i(x):
    return jnp.full((B,), x, dtype=_i32)


def _edge_loop(biota, pack_v, bonds_v, allsites_v, wbuf, bias_v, acc_ref):
    def body(t, carry):
        pk = pack_v[pl.ds(t * B, B)]
        soff = [pk[g] for g in range(G)]
        roff = [pk[G + g] for g in range(G)]
        accw = [pk[2 * G + g] for g in range(G)]
        bs = biota * pk[6]
        br = biota * pk[7]
        w0 = pk[8]
        b0 = pk[9]
        aw0 = pk[10]
        ab0 = pk[11]

        accs = tuple(bias_v[pl.ds((b0 + o) * B, B)]
                     for _ in range(G) for o in range(MSG))

        UNR = 1

        def gather_sub(col0, offs, bvec):
            def fb(ff, accs):
                accs = list(accs)
                for u in range(UNR):
                    f = ff * UNR + u
                    vs = [plsc.load_gather(
                        allsites_v, [bvec + _bcast_i(offs[g] + f)])
                        for g in range(G)]
                    wbase = (w0 + col0 + f) * B
                    for o in range(MSG):
                        w = wbuf[pl.ds(wbase + o * (F * B), B)]
                        for g in range(G):
                            accs[g * MSG + o] = accs[g * MSG + o] + w * vs[g]
                return tuple(accs)
            return fb

        accs = lax.fori_loop(0, IN // UNR, gather_sub(0, soff, bs), accs)
        accs = lax.fori_loop(0, IN // UNR, gather_sub(IN, roff, br), accs)
        eb = [(G * t + g) * (BOND * B) for g in range(G)]

        def bond_fb(ff, accs):
            accs = list(accs)
            for u in range(UNR):
                f = ff * UNR + u
                vs = [bonds_v[pl.ds(eb[g] + f * B, B)] for g in range(G)]
                wbase = (w0 + 2 * IN + f) * B
                for o in range(MSG):
                    w = wbuf[pl.ds(wbase + o * (F * B), B)]
                    for g in range(G):
                        accs[g * MSG + o] = accs[g * MSG + o] + w * vs[g]
            return tuple(accs)

        accs = lax.fori_loop(0, BOND // UNR, bond_fb, accs)

        for g in range(G):
            acc = [_leaky(accs[g * MSG + o]) for o in range(MSG)]
            s = bias_v[pl.ds(ab0 * B, B)]
            for o in range(MSG):
                s = s + bias_v[pl.ds((aw0 + o) * B, B)] * acc[o]
            att = _sigmoid(s)
            for o in range(MSG):
                plsc.addupdate_scatter(
                    acc_ref, [_bcast_i(accw[g] + o * B) + biota],
                    att * acc[o])
        return carry

    plsc.parallel_loop(0, EALL // G, step=1, carry=jnp.int32(0))(body)


def _mlp_site_loop(n_sites, biota, base, x_base, x_stride, m_off, p_off,
                   two_blocks, allsites_v, sh_red, xfeat_v, wbuf, w1_0, w2_0,
                   bias_v, b1_0, b2_0, nfeat, osites_v):
    bx = biota * x_stride

    def site_body(j, carry):
        n = base + j
        for k in range(IN):
            xfeat_v[pl.ds((_XF_X0 + k) * B, B)] = plsc.load_gather(
                allsites_v, [bx + _bcast_i(x_base + n * IN + k)])
        pltpu.sync_copy(sh_red.at[pl.ds((m_off + n * MSG) * B, MSG * B)],
                        xfeat_v.at[pl.ds(_XF_M0 * B, MSG * B)])
        if two_blocks:
            pltpu.sync_copy(sh_red.at[pl.ds((p_off + n * MSG) * B, MSG * B)],
                            xfeat_v.at[pl.ds(_XF_P0 * B, MSG * B)])

        for half in range(HID // MSG):
            accs = tuple(bias_v[pl.ds((b1_0 + half * MSG + o) * B, B)]
                         for o in range(MSG))

            def l1_body(k, accs):
                xk = xfeat_v[pl.ds(k * B, B)]
                accs = list(accs)
                for o in range(MSG):
                    w = wbuf[pl.ds(
                        (w1_0 + (half * MSG + o) * nfeat + k) * B, B)]
                    accs[o] = accs[o] + w * xk
                return tuple(accs)

            accs = lax.fori_loop(0, nfeat, l1_body, accs)
            for jj in range(MSG):
                xfeat_v[pl.ds((_XF_H0 + half * MSG + jj) * B, B)] = _leaky(
                    accs[jj])

        outs = tuple(bias_v[pl.ds((b2_0 + o) * B, B)] for o in range(OUT))

        def l2_body(jh, outs):
            hv = xfeat_v[pl.ds((_XF_H0 + jh) * B, B)]
            outs = list(outs)
            for o in range(OUT):
                w = wbuf[pl.ds((w2_0 + o * HID + jh) * B, B)]
                outs[o] = outs[o] + w * hv
            return tuple(outs)

        outs = lax.fori_loop(0, HID, l2_body, outs)
        for o in range(OUT):
            osites_v[pl.ds((j * OUT + o) * B, B)] = (
                xfeat_v[pl.ds((_XF_X0 + o) * B, B)] + _leaky(outs[o]))
        return carry

    lax.fori_loop(0, n_sites, site_body, 0)


def _sc_body(sites_h, sitesp_h, bonds_h, pack_h, wall_h, zeros_h,
             sites_out_h, sites_p_out_h,
             allsites_v, bonds_v, pack_v, wbuf, bias_v,
             acc_v, redin_v, redout_v,
             xfeat_v, osites_v, ositesp_v,
             sh_slots, sh_red):
    sid = lax.axis_index("s")
    biota = lax.iota(_i32, B)

    pltpu.sync_copy(sites_h, allsites_v.at[pl.ds(_S_BASE, B * N * IN)])
    pltpu.sync_copy(sitesp_h, allsites_v.at[pl.ds(_SP_BASE, B * NP * IN)])
    pltpu.sync_copy(pack_h.at[pl.ds(sid * (EALL // G) * B, (EALL // G) * B)],
                    pack_v)
    pltpu.sync_copy(wall_h.at[pl.ds(0, _WEDGE_ROWS * B)],
                    wbuf.at[pl.ds(0, _WEDGE_ROWS * B)])
    pltpu.sync_copy(wall_h.at[pl.ds(_BIAS0 * B, _BIAS_ROWS * B)], bias_v)
    pltpu.sync_copy(zeros_h, acc_v)
    pltpu.sync_copy(bonds_h.at[pl.ds(sid * EC * BOND * B, EC * BOND * B)],
                    bonds_v.at[pl.ds(0, EC * BOND * B)])
    pltpu.sync_copy(
        bonds_h.at[pl.ds((E + sid * EC_PS) * BOND * B, EC_PS * BOND * B)],
        bonds_v.at[pl.ds(EC * BOND * B, EC_PS * BOND * B)])
    pltpu.sync_copy(
        bonds_h.at[pl.ds((E + EPS + sid * EC_SP) * BOND * B,
                         EC_SP * BOND * B)],
        bonds_v.at[pl.ds((EC + EC_PS) * BOND * B, EC_SP * BOND * B)])

    _edge_loop(biota, pack_v, bonds_v, allsites_v, wbuf, bias_v, acc_v)

    for h in range(_NRED):
        pltpu.sync_copy(acc_v.at[pl.ds(h * _HW, _HW)],
                        sh_slots.at[pl.ds(sid * _HW, _HW)])
        plsc.subcore_barrier()
        for k in range(NS):
            pltpu.sync_copy(
                sh_slots.at[pl.ds(k * _HW + sid * _CHUNK, _CHUNK)],
                redin_v.at[pl.ds(k * _CHUNK, _CHUNK)])

        def red_body(r, carry):
            s = redin_v[pl.ds(r * B, B)]
            for k in range(1, NS):
                s = s + redin_v[pl.ds(k * _CHUNK + r * B, B)]
            redout_v[pl.ds(r * B, B)] = s
            return carry

        plsc.parallel_loop(0, _CHUNK // B, step=1,
                           carry=jnp.int32(0))(red_body)
        pltpu.sync_copy(redout_v,
                        sh_red.at[pl.ds(h * _HW + sid * _CHUNK, _CHUNK)])
        plsc.subcore_barrier()

    pltpu.sync_copy(wall_h.at[pl.ds(_WN_IN0 * B, _WN_ROWS * B)],
                    wbuf.at[pl.ds(0, _WN_ROWS * B)])
    _mlp_site_loop(SPT, biota, sid * SPT, _S_BASE, N * IN,
                   _ACC_M0, _ACC_PS0, True, allsites_v, sh_red, xfeat_v,
                   wbuf, _WN1_0, _WN2_0, bias_v, _BN1_0, _BN2_0,
                   IN + 2 * MSG, osites_v)
    pltpu.sync_copy(osites_v,
                    sites_out_h.at[pl.ds(sid * SPT * OUT * B, SPT * OUT * B)])

    @pl.when(sid < NP)
    def _():
        pltpu.sync_copy(wall_h.at[pl.ds(_WP_IN0 * B, _WP_ROWS * B)],
                        wbuf.at[pl.ds(0, _WP_ROWS * B)])
        _mlp_site_loop(1, biota, sid, _SP_BASE, NP * IN,
                       _ACC_SP0, _ACC_SP0, False, allsites_v, sh_red,
                       xfeat_v, wbuf, _WP1_0, _WP2_0, bias_v,
                       _BP1_0, _BP2_0, IN + MSG, ositesp_v)
        pltpu.sync_copy(ositesp_v,
                        sites_p_out_h.at[pl.ds(sid * OUT * B, OUT * B)])


_mesh = plsc.VectorSubcoreMesh(core_axis_name="c", subcore_axis_name="s",
                               num_cores=1)

_sc_call = functools.partial(
    pl.kernel,
    out_type=[
        jax.ShapeDtypeStruct((N * OUT * B,), _f32),
        jax.ShapeDtypeStruct((NP * OUT * B,), _f32),
    ],
    mesh=_mesh,
    compiler_params=pltpu.CompilerParams(needs_layout_passes=False),
    scratch_types=[
        pltpu.VMEM((B * (N + NP) * IN,), _f32),
        pltpu.VMEM((EALL * BOND * B,), _f32),
        pltpu.VMEM(((EALL // G) * B,), _i32),
        pltpu.VMEM((_WEDGE_ROWS * B,), _f32),
        pltpu.VMEM((_BIAS_ROWS * B,), _f32),
        pltpu.VMEM((_ACC_W,), _f32),
        pltpu.VMEM((NS * _CHUNK,), _f32),
        pltpu.VMEM((_CHUNK,), _f32),
        pltpu.VMEM((_XF_ROWS * B,), _f32),
        pltpu.VMEM((SPT * OUT * B,), _f32),
        pltpu.VMEM((OUT * B,), _f32),
        pltpu.VMEM_SHARED((NS * _HW,), _f32),
        pltpu.VMEM_SHARED((_ACC_W,), _f32),
    ],
)(_sc_body)


def _seg_pack(i1, i2, s_base, r_base, acc_base, stride_s, stride_r,
              w0, b0, aw0, ab0):
    n = i1.shape[0] // G
    c = [s_base + i1.reshape(-1, G) * IN,
         r_base + i2.reshape(-1, G) * IN,
         (acc_base + i2.reshape(-1, G) * MSG) * B]
    const = jnp.broadcast_to(
        jnp.array([stride_s, stride_r, w0, b0, aw0, ab0]
                  + [0] * (B - 3 * G - 6), _i32), (n, B - 3 * G))
    p = jnp.concatenate(c + [const], axis=1)
    return p.reshape(NS, n // NS, B)


@jax.jit
def _run(sites, bonds, sites_p, bonds_sp, bonds_ps,
         idx1, idx2, idx1_sp, idx2_sp, idx1_ps, idx2_ps,
         W_msg, b_msg, aW_msg, ab_msg,
         W_sp, b_sp, aW_sp, ab_sp,
         W_ps, b_ps, aW_ps, ab_ps,
         Wn1, bn1, Wn2, bn2, Wp1, bp1, Wp2, bp2):
    wall = jnp.concatenate([
        W_msg.reshape(-1), W_ps.reshape(-1), W_sp.reshape(-1),
        Wn1.reshape(-1), Wn2.reshape(-1),
        Wp1.reshape(-1), Wp2.reshape(-1),
        b_msg, aW_msg.reshape(-1), ab_msg,
        b_ps, aW_ps.reshape(-1), ab_ps,
        b_sp, aW_sp.reshape(-1), ab_sp,
        bn1, bn2, bp1, bp2])
    bcast = lambda v: jnp.broadcast_to(
        v.reshape(-1, 1), (v.shape[0], B)).reshape(-1)
    bonds_all = jnp.concatenate([bonds, bonds_ps, bonds_sp],
                                axis=1).transpose(1, 2, 0).reshape(-1)
    pack = jnp.concatenate([
        _seg_pack(idx1, idx2, _S_BASE, _S_BASE, _ACC_M0, N * IN, N * IN,
                  _W_MSG0, _B_MSG, _AW_MSG, _AB_MSG),
        _seg_pack(idx1_ps, idx2_ps, _SP_BASE, _S_BASE, _ACC_PS0, NP * IN,
                  N * IN, _W_PS0, _B_PS, _AW_PS, _AB_PS),
        _seg_pack(idx1_sp, idx2_sp, _S_BASE, _SP_BASE, _ACC_SP0, N * IN,
                  NP * IN, _W_SP0, _B_SP, _AW_SP, _AB_SP),
    ], axis=1).reshape(-1)
    return _sc_call(
        sites.reshape(-1), sites_p.reshape(-1), bonds_all, pack,
        bcast(wall), jnp.zeros((_ACC_W,), _f32))


def kernel(sites, bonds, sites_p, bonds_sp, bonds_ps,
           idx1, idx2, idx1_sp, idx2_sp, idx1_ps, idx2_ps,
           W_msg, b_msg, aW_msg, ab_msg,
           W_sp, b_sp, aW_sp, ab_sp,
           W_ps, b_ps, aW_ps, ab_ps,
           Wn1, bn1, Wn2, bn2, Wp1, bp1, Wp2, bp2):
    i32 = lambda x: x.astype(_i32)
    sites_new_t, sites_p_new_t = _run(
        sites, bonds, sites_p, bonds_sp, bonds_ps,
        i32(idx1), i32(idx2), i32(idx1_sp), i32(idx2_sp),
        i32(idx1_ps), i32(idx2_ps),
        W_msg, b_msg, aW_msg, ab_msg,
        W_sp, b_sp, aW_sp, ab_sp,
        W_ps, b_ps, aW_ps, ab_ps,
        Wn1, bn1, Wn2, bn2, Wp1, bp1, Wp2, bp2)
    sites_new = sites_new_t.reshape(N, OUT, B).transpose(2, 0, 1)
    sites_p_new = sites_p_new_t.reshape(NP, OUT, B).transpose(2, 0, 1)
    return (sites_new, bonds, sites_p_new, bonds_sp, bonds_ps)
